# baseline (speedup 1.0000x reference)
import functools

import jax
import jax.numpy as jnp
from jax import lax
from jax.experimental import pallas as pl
from jax.experimental.pallas import tpu as pltpu
from jax.experimental.pallas import tpu_sc as plsc

_N = 50000
_D = 128
_R = 15
_C = 12544
_NCHUNK = 4
_NPAD = _C * _NCHUNK
_B = 128
_HB = 64
_DUMMY = _C
_ACC_ROWS = _C + 8
_BN = 400
_NB = _N // _BN
_FPT = _ACC_ROWS // 16
_EPS = 1e-5


def _gn(x, g, b):
    m = jnp.mean(x, axis=-1, keepdims=True)
    v = jnp.mean((x - m) ** 2, axis=-1, keepdims=True)
    return (x - m) * lax.rsqrt(v + _EPS) * g + b


def _input_body(ctrs_ref, feats_ref, w1, b1, w2, g1, bb1, w3, b3, w4, g2, bb2,
                out_ref):
    x = ctrs_ref[...]
    h = x[:, 0:1] * w1[0:1, :] + x[:, 1:2] * w1[1:2, :] + b1[...]
    h = jnp.maximum(h, 0.0)
    h = jnp.dot(h, w2[...], preferred_element_type=jnp.float32)
    h = _gn(h, g1[...], bb1[...])
    y = feats_ref[...]
    s = y[:, 0:1] * w3[0:1, :] + y[:, 1:2] * w3[1:2, :] + b3[...]
    s = jnp.maximum(s, 0.0)
    s = jnp.dot(s, w4[...], preferred_element_type=jnp.float32)
    s = _gn(s, g2[...], bb2[...])
    out_ref[...] = jnp.maximum(h + s, 0.0)


def _input_layer(ctrs, feats, W_in1, b_in1, W_in2, g_in, b_in,
                 W_seg1, b_seg1, W_seg2, g_seg, b_seg):
    row = pl.BlockSpec((_BN, 2), lambda i: (i, 0))
    full = lambda shape: pl.BlockSpec(shape, lambda i: (0,) * len(shape))
    return pl.pallas_call(
        _input_body,
        grid=(_NB,),
        in_specs=[row, row,
                  full((2, _D)), full((1, _D)), full((_D, _D)),
                  full((1, _D)), full((1, _D)),
                  full((2, _D)), full((1, _D)), full((_D, _D)),
                  full((1, _D)), full((1, _D))],
        out_specs=pl.BlockSpec((_BN, _D), lambda i: (i, 0)),
        out_shape=jax.ShapeDtypeStruct((_N, _D), jnp.float32),
    )(ctrs, feats, W_in1, b_in1.reshape(1, _D), W_in2,
      g_in.reshape(1, _D), b_in.reshape(1, _D),
      W_seg1, b_seg1.reshape(1, _D), W_seg2,
      g_seg.reshape(1, _D), b_seg.reshape(1, _D))


def _mm_body(feat_ref, w_ref, out_ref):
    x = feat_ref[...].astype(jnp.bfloat16)
    for r in range(_R):
        out_ref[r] = jnp.dot(x, w_ref[r], preferred_element_type=jnp.float32)


def _matmul15(feat, W_all_i):
    return pl.pallas_call(
        _mm_body,
        grid=(_NB,),
        in_specs=[pl.BlockSpec((_BN, _D), lambda i: (i, 0)),
                  pl.BlockSpec((_R, _D, _D), lambda i: (0, 0, 0))],
        out_specs=pl.BlockSpec((_R, _BN, _D), lambda i: (0, i, 0)),
        out_shape=jax.ShapeDtypeStruct((_R, _N, _D), jnp.float32),
    )(feat, W_all_i)


@functools.partial(
    pl.kernel,
    out_type=jax.ShapeDtypeStruct((_NPAD, _D), jnp.float32),
    mesh=plsc.VectorSubcoreMesh(core_axis_name="c", subcore_axis_name="s"),
    scratch_types=[
        pltpu.VMEM_SHARED((_ACC_ROWS, _D), jnp.float32),
        pltpu.VMEM((2, _HB), jnp.int32),
        pltpu.VMEM((2, _HB), jnp.int32),
        pltpu.VMEM((2, _HB), jnp.int32),
        pltpu.VMEM((_B, _D), jnp.float32),
        pltpu.VMEM((16,), jnp.int32),
        pltpu.SemaphoreType.DMA,
        pltpu.SemaphoreType.DMA,
    ],
)
def _sc_scatter(P_hbm, u_hbm, s_hbm, offs_hbm, z_hbm, out_hbm,
                accum, uv, sv, lv, rows, offs_v, sem0, sem1):
    cid = lax.axis_index("c")
    sid = lax.axis_index("s")
    pltpu.sync_copy(offs_hbm, offs_v)
    offs = offs_v[...]
    for c_local in range(2):
        chunk = cid * 2 + c_local
        base = chunk * _C
        lo_e = jnp.where(cid == 0, offs[c_local], offs[c_local + 2])
        hi_e = jnp.where(cid == 0, offs[c_local + 1], offs[c_local + 3])
        lo = lo_e // _HB
        hi = (hi_e + _HB - 1) // _HB
        pltpu.sync_copy(z_hbm, rows)
        for j in range(7):
            k = sid * 7 + j

            @pl.when(k < _C // _B)
            def _():
                pltpu.sync_copy(rows, accum.at[pl.ds(k * _B, _B)])

        plsc.subcore_barrier()
        n_own = (jnp.maximum(hi - lo - sid, 0) + 15) // 16

        def _fetch(h, t):
            e0 = (lo + sid + 16 * t) * _HB
            pltpu.sync_copy(u_hbm.at[pl.ds(e0, _HB)], uv.at[h])
            pltpu.sync_copy(s_hbm.at[pl.ds(e0, _HB)], sv.at[h])

        def _start(h, sem):
            pltpu.async_copy(P_hbm.at[sv.at[h]],
                             rows.at[pl.ds(h * _HB, _HB)], sem)

        def _wait(h, sem):
            pltpu.make_async_copy(P_hbm.at[sv.at[h]],
                                  rows.at[pl.ds(h * _HB, _HB)], sem).wait()

        def _mask_scatter(h):
            for j in range(_HB // 16):
                u16 = uv[h, pl.ds(j * 16, 16)]
                lu = u16 - base
                ok = (lu >= 0) & (lu < _C)
                lv[h, pl.ds(j * 16, 16)] = jnp.where(ok, lu, _DUMMY)
            pltpu.sync_copy(rows.at[pl.ds(h * _HB, _HB)], accum.at[lv.at[h]],
                            add=True)

        @pl.when(n_own > 0)
        def _():
            _fetch(0, 0)
            _start(0, sem0)

        def _pair(p, carry):
            t1 = 2 * p + 1

            @pl.when(t1 < n_own)
            def _():
                _fetch(1, t1)
                _start(1, sem1)

            _wait(0, sem0)
            _mask_scatter(0)

            @pl.when(t1 + 1 < n_own)
            def _():
                _fetch(0, t1 + 1)
                _start(0, sem0)

            @pl.when(t1 < n_own)
            def _():
                _wait(1, sem1)
                _mask_scatter(1)

            return carry

        lax.fori_loop(0, (n_own + 1) // 2, _pair, 0)
        plsc.subcore_barrier()
        r0 = sid * _FPT
        for off, sz in ((0, 128), (128, 128), (256, 128), (384, 128),
                        (512, 128), (640, 128), (768, 16)):
            pltpu.sync_copy(accum.at[pl.ds(r0 + off, sz)],
                            out_hbm.at[pl.ds(base + r0 + off, sz)])
        plsc.subcore_barrier()


def _norm_body(p0_ref, sc_ref, res_ref, gn_ref, bn_ref, w2_ref, g2_ref,
               b2_ref, out_ref):
    temp = p0_ref[...] + sc_ref[...]
    t = jnp.maximum(_gn(temp, gn_ref[...], bn_ref[...]), 0.0)
    h = jnp.dot(t, w2_ref[...], preferred_element_type=jnp.float32)
    h = _gn(h, g2_ref[...], b2_ref[...])
    out_ref[...] = jnp.maximum(h + res_ref[...], 0.0)


def _norm_layer(P, scat, res, g_n, b_n, W2, g2, b2):
    row = pl.BlockSpec((_BN, _D), lambda i: (i, 0))
    full = lambda shape: pl.BlockSpec(shape, lambda i: (0,) * len(shape))
    return pl.pallas_call(
        _norm_body,
        grid=(_NB,),
        in_specs=[row, row, row,
                  full((1, _D)), full((1, _D)), full((_D, _D)),
                  full((1, _D)), full((1, _D))],
        out_specs=row,
        out_shape=jax.ShapeDtypeStruct((_N, _D), jnp.float32),
    )(P, scat, res, g_n.reshape(1, _D), b_n.reshape(1, _D), W2,
      g2.reshape(1, _D), b2.reshape(1, _D))


def kernel(ctrs, feats, pre_u, pre_v, suc_u, suc_v, left_u, left_v, right_u,
           right_v, W_in1, b_in1, W_in2, g_in, b_in, W_seg1, b_seg1, W_seg2,
           g_seg, b_seg, W_ctr, W_pre, W_suc, W_left, W_right, g_norm, b_norm,
           W_ctr2, g_ctr2, b_ctr2):
    ns = pre_u.shape[0]
    koff = jnp.arange(ns, dtype=jnp.int32)[:, None]
    src = jnp.concatenate([
        (pre_v.astype(jnp.int32) + (koff + 1) * _N).reshape(-1),
        (suc_v.astype(jnp.int32) + (koff + 1 + ns) * _N).reshape(-1),
        left_v.astype(jnp.int32) + (1 + 2 * ns) * _N,
        right_v.astype(jnp.int32) + (2 + 2 * ns) * _N,
    ])
    u_all = jnp.concatenate([
        pre_u.reshape(-1), suc_u.reshape(-1), left_u, right_u,
    ]).astype(jnp.int32)
    chunk = u_all // _C
    ranks = []
    cnts = []
    for c in range(_NCHUNK):
        m = (chunk == c).astype(jnp.int32)
        r = jnp.cumsum(m)
        ranks.append(r - 1)
        cnts.append(r[-1])
    off = jnp.concatenate([jnp.zeros((1,), jnp.int32),
                           jnp.cumsum(jnp.stack(cnts))]).astype(jnp.int32)
    pos = jnp.zeros_like(u_all)
    for c in range(_NCHUNK):
        pos = jnp.where(chunk == c, off[c] + ranks[c], pos)
    zero_e = jnp.zeros_like(u_all)
    u_srt = zero_e.at[pos].set(u_all, unique_indices=True,
                               mode='promise_in_bounds')
    s_srt = zero_e.at[pos].set(src, unique_indices=True,
                               mode='promise_in_bounds')
    offs16 = jnp.concatenate(
        [off, jnp.full((11,), u_all.shape[0], jnp.int32)])
    zrows = jnp.zeros((_B, _D), jnp.float32)

    W_all = jnp.concatenate([
        W_ctr[:, None], jnp.transpose(W_pre, (1, 0, 2, 3)),
        jnp.transpose(W_suc, (1, 0, 2, 3)), W_left[:, None],
        W_right[:, None],
    ], axis=1).astype(jnp.bfloat16)

    feat = _input_layer(ctrs, feats, W_in1, b_in1, W_in2, g_in, b_in,
                        W_seg1, b_seg1, W_seg2, g_seg, b_seg)
    res = feat
    for i in range(4):
        P = _matmul15(feat, W_all[i]).reshape(_R * _N, _D)
        scat = _sc_scatter(P, u_srt, s_srt, offs16, zrows)
        feat = _norm_layer(P, scat, res, g_norm[i], b_norm[i], W_ctr2[i],
                           g_ctr2[i], b_ctr2[i])
        res = feat
    return feat


# speedup vs baseline: 2.2536x; 2.2536x over previous
import functools

import jax
import jax.numpy as jnp
from jax import lax
from jax.experimental import pallas as pl
from jax.experimental.pallas import tpu as pltpu
from jax.experimental.pallas import tpu_sc as plsc

_N = 50000
_D = 128
_R = 15
_C = 12544
_NCHUNK = 4
_NPAD = _C * _NCHUNK
_B = 128
_HB = 64
_DUMMY = _C
_ACC_ROWS = _C + 8
_BN = 400
_NB = _N // _BN
_FPT = _ACC_ROWS // 16
_EPS = 1e-5


def _gn(x, g, b):
    m = jnp.mean(x, axis=-1, keepdims=True)
    v = jnp.mean((x - m) ** 2, axis=-1, keepdims=True)
    return (x - m) * lax.rsqrt(v + _EPS) * g + b


def _input_body(ctrs_ref, feats_ref, w1, b1, w2, g1, bb1, w3, b3, w4, g2, bb2,
                out_ref):
    x = ctrs_ref[...]
    h = x[:, 0:1] * w1[0:1, :] + x[:, 1:2] * w1[1:2, :] + b1[...]
    h = jnp.maximum(h, 0.0)
    h = jnp.dot(h, w2[...], preferred_element_type=jnp.float32)
    h = _gn(h, g1[...], bb1[...])
    y = feats_ref[...]
    s = y[:, 0:1] * w3[0:1, :] + y[:, 1:2] * w3[1:2, :] + b3[...]
    s = jnp.maximum(s, 0.0)
    s = jnp.dot(s, w4[...], preferred_element_type=jnp.float32)
    s = _gn(s, g2[...], bb2[...])
    out_ref[...] = jnp.maximum(h + s, 0.0)


def _input_layer(ctrs, feats, W_in1, b_in1, W_in2, g_in, b_in,
                 W_seg1, b_seg1, W_seg2, g_seg, b_seg):
    row = pl.BlockSpec((_BN, 2), lambda i: (i, 0))
    full = lambda shape: pl.BlockSpec(shape, lambda i: (0,) * len(shape))
    return pl.pallas_call(
        _input_body,
        grid=(_NB,),
        in_specs=[row, row,
                  full((2, _D)), full((1, _D)), full((_D, _D)),
                  full((1, _D)), full((1, _D)),
                  full((2, _D)), full((1, _D)), full((_D, _D)),
                  full((1, _D)), full((1, _D))],
        out_specs=pl.BlockSpec((_BN, _D), lambda i: (i, 0)),
        out_shape=jax.ShapeDtypeStruct((_N, _D), jnp.float32),
    )(ctrs, feats, W_in1, b_in1.reshape(1, _D), W_in2,
      g_in.reshape(1, _D), b_in.reshape(1, _D),
      W_seg1, b_seg1.reshape(1, _D), W_seg2,
      g_seg.reshape(1, _D), b_seg.reshape(1, _D))


def _mm_body(feat_ref, w_ref, out_ref):
    x = feat_ref[...].astype(jnp.bfloat16)
    for r in range(_R):
        out_ref[r] = jnp.dot(x, w_ref[r], preferred_element_type=jnp.float32)


def _matmul15(feat, W_all_i):
    return pl.pallas_call(
        _mm_body,
        grid=(_NB,),
        in_specs=[pl.BlockSpec((_BN, _D), lambda i: (i, 0)),
                  pl.BlockSpec((_R, _D, _D), lambda i: (0, 0, 0))],
        out_specs=pl.BlockSpec((_R, _BN, _D), lambda i: (0, i, 0)),
        out_shape=jax.ShapeDtypeStruct((_R, _N, _D), jnp.float32),
    )(feat, W_all_i)


@functools.partial(
    pl.kernel,
    out_type=jax.ShapeDtypeStruct((_NPAD, _D), jnp.float32),
    mesh=plsc.VectorSubcoreMesh(core_axis_name="c", subcore_axis_name="s"),
    scratch_types=[
        pltpu.VMEM_SHARED((_ACC_ROWS, _D), jnp.float32),
        pltpu.VMEM((2, _HB), jnp.int32),
        pltpu.VMEM((2, _HB), jnp.int32),
        pltpu.VMEM((2, _HB), jnp.int32),
        pltpu.VMEM((_B, _D), jnp.float32),
        pltpu.VMEM((16,), jnp.int32),
        pltpu.SemaphoreType.DMA,
        pltpu.SemaphoreType.DMA,
    ],
)
def _sc_scatter(P_hbm, u_hbm, s_hbm, offs_hbm, z_hbm, out_hbm,
                accum, uv, sv, lv, rows, offs_v, sem0, sem1):
    cid = lax.axis_index("c")
    sid = lax.axis_index("s")
    pltpu.sync_copy(offs_hbm, offs_v)
    offs = offs_v[...]
    for c_local in range(2):
        chunk = cid * 2 + c_local
        base = chunk * _C
        lo_e = jnp.where(cid == 0, offs[c_local], offs[c_local + 2])
        hi_e = jnp.where(cid == 0, offs[c_local + 1], offs[c_local + 3])
        lo = lo_e // _HB
        hi = (hi_e + _HB - 1) // _HB
        pltpu.sync_copy(z_hbm, rows)
        for j in range(7):
            k = sid * 7 + j

            @pl.when(k < _C // _B)
            def _():
                pltpu.sync_copy(rows, accum.at[pl.ds(k * _B, _B)])

        plsc.subcore_barrier()
        n_own = (jnp.maximum(hi - lo - sid, 0) + 15) // 16

        def _fetch(h, t):
            e0 = (lo + sid + 16 * t) * _HB
            pltpu.sync_copy(u_hbm.at[pl.ds(e0, _HB)], uv.at[h])
            pltpu.sync_copy(s_hbm.at[pl.ds(e0, _HB)], sv.at[h])

        def _start(h, sem):
            pltpu.async_copy(P_hbm.at[sv.at[h]],
                             rows.at[pl.ds(h * _HB, _HB)], sem)

        def _wait(h, sem):
            pltpu.make_async_copy(P_hbm.at[sv.at[h]],
                                  rows.at[pl.ds(h * _HB, _HB)], sem).wait()

        def _mask_scatter(h):
            for j in range(_HB // 16):
                u16 = uv[h, pl.ds(j * 16, 16)]
                lu = u16 - base
                ok = (lu >= 0) & (lu < _C)
                lv[h, pl.ds(j * 16, 16)] = jnp.where(ok, lu, _DUMMY)
            pltpu.sync_copy(rows.at[pl.ds(h * _HB, _HB)], accum.at[lv.at[h]],
                            add=True)

        @pl.when(n_own > 0)
        def _():
            _fetch(0, 0)
            _start(0, sem0)

        def _pair(p, carry):
            t1 = 2 * p + 1

            @pl.when(t1 < n_own)
            def _():
                _fetch(1, t1)
                _start(1, sem1)

            _wait(0, sem0)
            _mask_scatter(0)

            @pl.when(t1 + 1 < n_own)
            def _():
                _fetch(0, t1 + 1)
                _start(0, sem0)

            @pl.when(t1 < n_own)
            def _():
                _wait(1, sem1)
                _mask_scatter(1)

            return carry

        lax.fori_loop(0, (n_own + 1) // 2, _pair, 0)
        plsc.subcore_barrier()
        r0 = sid * _FPT
        for off, sz in ((0, 128), (128, 128), (256, 128), (384, 128),
                        (512, 128), (640, 128), (768, 16)):
            pltpu.sync_copy(accum.at[pl.ds(r0 + off, sz)],
                            out_hbm.at[pl.ds(base + r0 + off, sz)])
        plsc.subcore_barrier()


def _norm_body(p0_ref, sc_ref, res_ref, gn_ref, bn_ref, w2_ref, g2_ref,
               b2_ref, out_ref):
    temp = p0_ref[...] + sc_ref[...]
    t = jnp.maximum(_gn(temp, gn_ref[...], bn_ref[...]), 0.0)
    h = jnp.dot(t, w2_ref[...], preferred_element_type=jnp.float32)
    h = _gn(h, g2_ref[...], b2_ref[...])
    out_ref[...] = jnp.maximum(h + res_ref[...], 0.0)


def _norm_layer(P, scat, res, g_n, b_n, W2, g2, b2):
    row = pl.BlockSpec((_BN, _D), lambda i: (i, 0))
    full = lambda shape: pl.BlockSpec(shape, lambda i: (0,) * len(shape))
    return pl.pallas_call(
        _norm_body,
        grid=(_NB,),
        in_specs=[row, row, row,
                  full((1, _D)), full((1, _D)), full((_D, _D)),
                  full((1, _D)), full((1, _D))],
        out_specs=row,
        out_shape=jax.ShapeDtypeStruct((_N, _D), jnp.float32),
    )(P, scat, res, g_n.reshape(1, _D), b_n.reshape(1, _D), W2,
      g2.reshape(1, _D), b2.reshape(1, _D))


def kernel(ctrs, feats, pre_u, pre_v, suc_u, suc_v, left_u, left_v, right_u,
           right_v, W_in1, b_in1, W_in2, g_in, b_in, W_seg1, b_seg1, W_seg2,
           g_seg, b_seg, W_ctr, W_pre, W_suc, W_left, W_right, g_norm, b_norm,
           W_ctr2, g_ctr2, b_ctr2):
    ns = pre_u.shape[0]
    koff = jnp.arange(ns, dtype=jnp.int32)[:, None]
    src = jnp.concatenate([
        (pre_v.astype(jnp.int32) + (koff + 1) * _N).reshape(-1),
        (suc_v.astype(jnp.int32) + (koff + 1 + ns) * _N).reshape(-1),
        left_v.astype(jnp.int32) + (1 + 2 * ns) * _N,
        right_v.astype(jnp.int32) + (2 + 2 * ns) * _N,
    ])
    u_all = jnp.concatenate([
        pre_u.reshape(-1), suc_u.reshape(-1), left_u, right_u,
    ]).astype(jnp.int32)
    ne = u_all.shape[0]
    key = ((u_all // _C) << 20) | jnp.arange(ne, dtype=jnp.int32)
    (ksort,) = lax.sort((key,), dimension=0, num_keys=1)
    idx = ksort & 0xFFFFF
    u_srt = u_all[idx]
    s_srt = src[idx]
    off = jnp.searchsorted(
        ksort, jnp.arange(5, dtype=jnp.int32) << 20).astype(jnp.int32)
    offs16 = jnp.concatenate([off, jnp.full((11,), ne, jnp.int32)])
    zrows = jnp.zeros((_B, _D), jnp.float32)

    W_all = jnp.concatenate([
        W_ctr[:, None], jnp.transpose(W_pre, (1, 0, 2, 3)),
        jnp.transpose(W_suc, (1, 0, 2, 3)), W_left[:, None],
        W_right[:, None],
    ], axis=1).astype(jnp.bfloat16)

    feat = _input_layer(ctrs, feats, W_in1, b_in1, W_in2, g_in, b_in,
                        W_seg1, b_seg1, W_seg2, g_seg, b_seg)
    res = feat
    for i in range(4):
        P = _matmul15(feat, W_all[i]).reshape(_R * _N, _D)
        scat = _sc_scatter(P, u_srt, s_srt, offs16, zrows)
        feat = _norm_layer(P, scat, res, g_norm[i], b_norm[i], W_ctr2[i],
                           g_ctr2[i], b_ctr2[i])
        res = feat
    return feat


# speedup vs baseline: 2.4407x; 1.0830x over previous
import functools

import jax
import jax.numpy as jnp
from jax import lax
from jax.experimental import pallas as pl
from jax.experimental.pallas import tpu as pltpu
from jax.experimental.pallas import tpu_sc as plsc

_N = 50000
_D = 128
_R = 15
_C = 12544
_NCHUNK = 4
_NPAD = _C * _NCHUNK
_B = 128
_HB = 64
_DUMMY = _C
_ACC_ROWS = _C + 8
_BN = 400
_NB = _N // _BN
_FPT = _ACC_ROWS // 16
_EPS = 1e-5


def _gn(x, g, b):
    m = jnp.mean(x, axis=-1, keepdims=True)
    v = jnp.mean((x - m) ** 2, axis=-1, keepdims=True)
    return (x - m) * lax.rsqrt(v + _EPS) * g + b


def _input_body(ctrs_ref, feats_ref, w1, b1, w2, g1, bb1, w3, b3, w4, g2, bb2,
                out_ref):
    x = ctrs_ref[...]
    h = x[:, 0:1] * w1[0:1, :] + x[:, 1:2] * w1[1:2, :] + b1[...]
    h = jnp.maximum(h, 0.0)
    h = jnp.dot(h, w2[...], preferred_element_type=jnp.float32)
    h = _gn(h, g1[...], bb1[...])
    y = feats_ref[...]
    s = y[:, 0:1] * w3[0:1, :] + y[:, 1:2] * w3[1:2, :] + b3[...]
    s = jnp.maximum(s, 0.0)
    s = jnp.dot(s, w4[...], preferred_element_type=jnp.float32)
    s = _gn(s, g2[...], bb2[...])
    out_ref[...] = jnp.maximum(h + s, 0.0)


def _input_layer(ctrs, feats, W_in1, b_in1, W_in2, g_in, b_in,
                 W_seg1, b_seg1, W_seg2, g_seg, b_seg):
    row = pl.BlockSpec((_BN, 2), lambda i: (i, 0))
    full = lambda shape: pl.BlockSpec(shape, lambda i: (0,) * len(shape))
    return pl.pallas_call(
        _input_body,
        grid=(_NB,),
        in_specs=[row, row,
                  full((2, _D)), full((1, _D)), full((_D, _D)),
                  full((1, _D)), full((1, _D)),
                  full((2, _D)), full((1, _D)), full((_D, _D)),
                  full((1, _D)), full((1, _D))],
        out_specs=pl.BlockSpec((_BN, _D), lambda i: (i, 0)),
        out_shape=jax.ShapeDtypeStruct((_N, _D), jnp.float32),
    )(ctrs, feats, W_in1, b_in1.reshape(1, _D), W_in2,
      g_in.reshape(1, _D), b_in.reshape(1, _D),
      W_seg1, b_seg1.reshape(1, _D), W_seg2,
      g_seg.reshape(1, _D), b_seg.reshape(1, _D))


def _mm_body(feat_ref, w_ref, out_ref):
    x = feat_ref[...].astype(jnp.bfloat16)
    for r in range(_R):
        out_ref[r] = jnp.dot(x, w_ref[r], preferred_element_type=jnp.float32)


def _matmul15(feat, W_all_i):
    return pl.pallas_call(
        _mm_body,
        grid=(_NB,),
        in_specs=[pl.BlockSpec((_BN, _D), lambda i: (i, 0)),
                  pl.BlockSpec((_R, _D, _D), lambda i: (0, 0, 0))],
        out_specs=pl.BlockSpec((_R, _BN, _D), lambda i: (0, i, 0)),
        out_shape=jax.ShapeDtypeStruct((_R, _N, _D), jnp.float32),
    )(feat, W_all_i)


@functools.partial(
    pl.kernel,
    out_type=jax.ShapeDtypeStruct((_NPAD, _D), jnp.float32),
    mesh=plsc.VectorSubcoreMesh(core_axis_name="c", subcore_axis_name="s"),
    scratch_types=[
        pltpu.VMEM_SHARED((_ACC_ROWS, _D), jnp.float32),
        pltpu.VMEM((2, 2, _HB), jnp.int32),
        pltpu.VMEM((2, _HB), jnp.int32),
        pltpu.VMEM((_B, _D), jnp.float32),
        pltpu.VMEM((16,), jnp.int32),
        pltpu.SemaphoreType.DMA,
        pltpu.SemaphoreType.DMA,
    ],
)
def _sc_scatter(P_hbm, us_hbm, offs_hbm, z_hbm, out_hbm,
                accum, usv, lv, rows, offs_v, sem0, sem1):
    cid = lax.axis_index("c")
    sid = lax.axis_index("s")
    pltpu.sync_copy(offs_hbm, offs_v)
    offs = offs_v[...]
    for c_local in range(2):
        chunk = cid * 2 + c_local
        base = chunk * _C
        lo_e = jnp.where(cid == 0, offs[c_local], offs[c_local + 2])
        hi_e = jnp.where(cid == 0, offs[c_local + 1], offs[c_local + 3])
        lo = lo_e // _HB
        hi = (hi_e + _HB - 1) // _HB
        pltpu.sync_copy(z_hbm, rows)
        for j in range(7):
            k = sid * 7 + j

            @pl.when(k < _C // _B)
            def _():
                pltpu.sync_copy(rows, accum.at[pl.ds(k * _B, _B)])

        plsc.subcore_barrier()
        n_own = (jnp.maximum(hi - lo - sid, 0) + 15) // 16

        def _fetch(h, t):
            b = lo + sid + 16 * t
            pltpu.sync_copy(us_hbm.at[b], usv.at[h])

        def _start(h, sem):
            pltpu.async_copy(P_hbm.at[usv.at[h, 1]],
                             rows.at[pl.ds(h * _HB, _HB)], sem)

        def _wait(h, sem):
            pltpu.make_async_copy(P_hbm.at[usv.at[h, 1]],
                                  rows.at[pl.ds(h * _HB, _HB)], sem).wait()

        def _mask_scatter(h):
            for j in range(_HB // 16):
                u16 = usv[h, 0, pl.ds(j * 16, 16)]
                lu = u16 - base
                ok = (lu >= 0) & (lu < _C)
                lv[h, pl.ds(j * 16, 16)] = jnp.where(ok, lu, _DUMMY)
            pltpu.sync_copy(rows.at[pl.ds(h * _HB, _HB)], accum.at[lv.at[h]],
                            add=True)

        @pl.when(n_own > 0)
        def _():
            _fetch(0, 0)
            _start(0, sem0)

        def _pair(p, carry):
            t1 = 2 * p + 1

            @pl.when(t1 < n_own)
            def _():
                _fetch(1, t1)
                _start(1, sem1)

            _wait(0, sem0)
            _mask_scatter(0)

            @pl.when(t1 + 1 < n_own)
            def _():
                _fetch(0, t1 + 1)
                _start(0, sem0)

            @pl.when(t1 < n_own)
            def _():
                _wait(1, sem1)
                _mask_scatter(1)

            return carry

        lax.fori_loop(0, (n_own + 1) // 2, _pair, 0)
        plsc.subcore_barrier()
        r0 = sid * _FPT
        for off, sz in ((0, 128), (128, 128), (256, 128), (384, 128),
                        (512, 128), (640, 128), (768, 16)):
            pltpu.sync_copy(accum.at[pl.ds(r0 + off, sz)],
                            out_hbm.at[pl.ds(base + r0 + off, sz)])
        plsc.subcore_barrier()


def _norm_body(p0_ref, sc_ref, res_ref, gn_ref, bn_ref, w2_ref, g2_ref,
               b2_ref, out_ref):
    temp = p0_ref[...] + sc_ref[...]
    t = jnp.maximum(_gn(temp, gn_ref[...], bn_ref[...]), 0.0)
    h = jnp.dot(t, w2_ref[...], preferred_element_type=jnp.float32)
    h = _gn(h, g2_ref[...], b2_ref[...])
    out_ref[...] = jnp.maximum(h + res_ref[...], 0.0)


def _norm_layer(P, scat, res, g_n, b_n, W2, g2, b2):
    row = pl.BlockSpec((_BN, _D), lambda i: (i, 0))
    full = lambda shape: pl.BlockSpec(shape, lambda i: (0,) * len(shape))
    return pl.pallas_call(
        _norm_body,
        grid=(_NB,),
        in_specs=[row, row, row,
                  full((1, _D)), full((1, _D)), full((_D, _D)),
                  full((1, _D)), full((1, _D))],
        out_specs=row,
        out_shape=jax.ShapeDtypeStruct((_N, _D), jnp.float32),
    )(P, scat, res, g_n.reshape(1, _D), b_n.reshape(1, _D), W2,
      g2.reshape(1, _D), b2.reshape(1, _D))


def kernel(ctrs, feats, pre_u, pre_v, suc_u, suc_v, left_u, left_v, right_u,
           right_v, W_in1, b_in1, W_in2, g_in, b_in, W_seg1, b_seg1, W_seg2,
           g_seg, b_seg, W_ctr, W_pre, W_suc, W_left, W_right, g_norm, b_norm,
           W_ctr2, g_ctr2, b_ctr2):
    ns = pre_u.shape[0]
    koff = jnp.arange(ns, dtype=jnp.int32)[:, None]
    src = jnp.concatenate([
        (pre_v.astype(jnp.int32) + (koff + 1) * _N).reshape(-1),
        (suc_v.astype(jnp.int32) + (koff + 1 + ns) * _N).reshape(-1),
        left_v.astype(jnp.int32) + (1 + 2 * ns) * _N,
        right_v.astype(jnp.int32) + (2 + 2 * ns) * _N,
    ])
    u_all = jnp.concatenate([
        pre_u.reshape(-1), suc_u.reshape(-1), left_u, right_u,
    ]).astype(jnp.int32)
    ne = u_all.shape[0]
    key = ((u_all // _C) << 20) | jnp.arange(ne, dtype=jnp.int32)
    (ksort,) = lax.sort((key,), dimension=0, num_keys=1)
    idx = ksort & 0xFFFFF
    us2 = jnp.stack([u_all[idx].reshape(-1, _HB),
                     src[idx].reshape(-1, _HB)], axis=1)
    off = jnp.searchsorted(
        ksort, jnp.arange(5, dtype=jnp.int32) << 20).astype(jnp.int32)
    offs16 = jnp.concatenate([off, jnp.full((11,), ne, jnp.int32)])
    zrows = jnp.zeros((_B, _D), jnp.float32)

    W_all = jnp.concatenate([
        W_ctr[:, None], jnp.transpose(W_pre, (1, 0, 2, 3)),
        jnp.transpose(W_suc, (1, 0, 2, 3)), W_left[:, None],
        W_right[:, None],
    ], axis=1).astype(jnp.bfloat16)

    feat = _input_layer(ctrs, feats, W_in1, b_in1, W_in2, g_in, b_in,
                        W_seg1, b_seg1, W_seg2, g_seg, b_seg)
    res = feat
    for i in range(4):
        P = _matmul15(feat, W_all[i]).reshape(_R * _N, _D)
        scat = _sc_scatter(P, us2, offs16, zrows)
        feat = _norm_layer(P, scat, res, g_norm[i], b_norm[i], W_ctr2[i],
                           g_ctr2[i], b_ctr2[i])
        res = feat
    return feat
